# initial kernel scaffold (unmeasured)
import jax
import jax.numpy as jnp
from jax import lax
from jax.experimental import pallas as pl
from jax.experimental.pallas import tpu as pltpu

N_DEV = 4
B, SQ, SKV, HQ, DH = 2, 128, 512, 4, 64
SKV_SHARD = SKV // N_DEV
DM = 512
SCALE = 0.125


def kernel(x, Wq, K_ext, V_ext, Wo):
    xb = x.astype(jnp.bfloat16)
    wq = Wq.astype(jnp.bfloat16)
    wo = Wo.astype(jnp.bfloat16)
    kt = jnp.transpose(K_ext, (0, 2, 1, 3)).astype(jnp.bfloat16)
    vt = jnp.transpose(V_ext, (0, 2, 1, 3)).astype(jnp.bfloat16)

    def body(x_ref, wq_ref, k_ref, v_ref, wo_ref, out_ref,
             k_all, v_all, ksend, krecv, vsend, vrecv):
        my_pos = lax.axis_index("i")
        left = (my_pos - 1) % N_DEV
        right = (my_pos + 1) % N_DEV

        barrier = pltpu.get_barrier_semaphore()
        for nbr in (left, right):
            pl.semaphore_signal(
                barrier, inc=1,
                device_id=(nbr,), device_id_type=pl.DeviceIdType.MESH,
            )
        pl.semaphore_wait(barrier, 2)

        for h in range(N_DEV - 1):
            src_k = k_ref if h == 0 else k_all.at[h - 1]
            src_v = v_ref if h == 0 else v_all.at[h - 1]
            rk = pltpu.make_async_remote_copy(
                src_ref=src_k, dst_ref=k_all.at[h],
                send_sem=ksend.at[h], recv_sem=krecv.at[h],
                device_id=(right,), device_id_type=pl.DeviceIdType.MESH,
            )
            rv = pltpu.make_async_remote_copy(
                src_ref=src_v, dst_ref=v_all.at[h],
                send_sem=vsend.at[h], recv_sem=vrecv.at[h],
                device_id=(right,), device_id_type=pl.DeviceIdType.MESH,
            )
            rk.start()
            rv.start()
            rk.wait()
            rv.wait()

        for b in range(B):
            q_b = lax.dot_general(
                x_ref[b], wq_ref[...], (((1,), (0,)), ((), ())),
                preferred_element_type=jnp.float32,
            )
            head_ctxs = []
            for h in range(HQ):
                q_bh = (q_b[:, h * DH:(h + 1) * DH] * SCALE).astype(jnp.bfloat16)
                sc_chunks = []
                for s in range(N_DEV):
                    k_chunk = k_ref[b, h] if s == 0 else k_all[s - 1, b, h]
                    sc = lax.dot_general(
                        q_bh, k_chunk, (((1,), (1,)), ((), ())),
                        preferred_element_type=jnp.float32,
                    )
                    origin = jnp.where(s == 0, my_pos, (my_pos - s - 1) % N_DEV) \
                        if False else ((my_pos - s) % N_DEV if s == 0 else (my_pos - s) % N_DEV)
                    sc_chunks.append((sc, s))
                masked = []
                for sc, s in sc_chunks:
                    origin = (my_pos - s) % N_DEV
                    qi = lax.broadcasted_iota(jnp.int32, (SQ, SKV_SHARD), 0)
                    kj = lax.broadcasted_iota(jnp.int32, (SQ, SKV_SHARD), 1) \
                        + origin * SKV_SHARD
                    mask = (jnp.abs(qi - kj) <= 128) | (kj < 32) | (qi < 32)
                    masked.append(jnp.where(mask, sc, -1e9))
                scores = jnp.concatenate(masked, axis=1)
                m = jnp.max(scores, axis=1, keepdims=True)
                w = jnp.exp(scores - m)
                w = w / jnp.sum(w, axis=1, keepdims=True)
                ctx = jnp.zeros((SQ, DH), jnp.float32)
                for s in range(N_DEV):
                    v_chunk = v_ref[b, h] if s == 0 else v_all[s - 1, b, h]
                    wb = w[:, s * SKV_SHARD:(s + 1) * SKV_SHARD].astype(jnp.bfloat16)
                    ctx = ctx + lax.dot_general(
                        wb, v_chunk, (((1,), (0,)), ((), ())),
                        preferred_element_type=jnp.float32,
                    )
                head_ctxs.append(ctx)
            ctx_b = jnp.concatenate(head_ctxs, axis=1).astype(jnp.bfloat16)
            out_ref[b] = lax.dot_general(
                ctx_b, wo_ref[...], (((1,), (0,)), ((), ())),
                preferred_element_type=jnp.float32,
            )

    return pl.pallas_call(
        body,
        out_shape=jax.ShapeDtypeStruct((B, SQ, DM), jnp.float32),
        in_specs=[pl.BlockSpec(memory_space=pltpu.VMEM)] * 5,
        out_specs=pl.BlockSpec(memory_space=pltpu.VMEM),
        scratch_shapes=[
            pltpu.VMEM((N_DEV - 1, B, HQ, SKV_SHARD, DH), jnp.bfloat16),
            pltpu.VMEM((N_DEV - 1, B, HQ, SKV_SHARD, DH), jnp.bfloat16),
            pltpu.SemaphoreType.DMA((N_DEV - 1,)),
            pltpu.SemaphoreType.DMA((N_DEV - 1,)),
            pltpu.SemaphoreType.DMA((N_DEV - 1,)),
            pltpu.SemaphoreType.DMA((N_DEV - 1,)),
        ],
        compiler_params=pltpu.CompilerParams(collective_id=0),
    )(xb, wq, kt, vt, wo)


# baseline (device time: 33605 ns/iter reference)
import jax
import jax.numpy as jnp
from jax import lax
from jax.experimental import pallas as pl
from jax.experimental.pallas import tpu as pltpu

N_DEV = 4
B, SQ, SKV, HQ, DH = 2, 128, 512, 4, 64
SKV_SHARD = SKV // N_DEV
DM = 512
SCALE = 0.125


def kernel(x, Wq, K_ext, V_ext, Wo):
    xb = x.astype(jnp.bfloat16)
    wq = Wq.astype(jnp.bfloat16)
    wo = Wo.astype(jnp.bfloat16)
    kt = jnp.transpose(K_ext, (0, 2, 1, 3)).astype(jnp.bfloat16)
    vt = jnp.transpose(V_ext, (0, 2, 1, 3)).astype(jnp.bfloat16)

    def body(x_ref, wq_ref, k_ref, v_ref, wo_ref, out_ref,
             k_all, v_all, ksend, krecv, vsend, vrecv):
        my_pos = lax.axis_index("i")
        left = (my_pos - 1) % N_DEV
        right = (my_pos + 1) % N_DEV

        barrier = pltpu.get_barrier_semaphore()
        for nbr in (left, right):
            pl.semaphore_signal(
                barrier, inc=1,
                device_id=(nbr,), device_id_type=pl.DeviceIdType.MESH,
            )
        pl.semaphore_wait(barrier, 2)

        for h in range(N_DEV - 1):
            src_k = k_ref if h == 0 else k_all.at[h - 1]
            src_v = v_ref if h == 0 else v_all.at[h - 1]
            rk = pltpu.make_async_remote_copy(
                src_ref=src_k, dst_ref=k_all.at[h],
                send_sem=ksend.at[h], recv_sem=krecv.at[h],
                device_id=(right,), device_id_type=pl.DeviceIdType.MESH,
            )
            rv = pltpu.make_async_remote_copy(
                src_ref=src_v, dst_ref=v_all.at[h],
                send_sem=vsend.at[h], recv_sem=vrecv.at[h],
                device_id=(right,), device_id_type=pl.DeviceIdType.MESH,
            )
            rk.start()
            rv.start()
            rk.wait()
            rv.wait()

        for b in range(B):
            q_b = lax.dot_general(
                x_ref[b], wq_ref[...], (((1,), (0,)), ((), ())),
                preferred_element_type=jnp.float32,
            )
            head_ctxs = []
            for h in range(HQ):
                q_bh = (q_b[:, h * DH:(h + 1) * DH] * SCALE).astype(jnp.bfloat16)
                masked = []
                for s in range(N_DEV):
                    k_chunk = k_ref[b, h] if s == 0 else k_all[s - 1, b, h]
                    sc = lax.dot_general(
                        q_bh, k_chunk, (((1,), (1,)), ((), ())),
                        preferred_element_type=jnp.float32,
                    )
                    origin = (my_pos - s) % N_DEV
                    qi = lax.broadcasted_iota(jnp.int32, (SQ, SKV_SHARD), 0)
                    kj = lax.broadcasted_iota(jnp.int32, (SQ, SKV_SHARD), 1) \
                        + origin * SKV_SHARD
                    mask = (jnp.abs(qi - kj) <= 128) | (kj < 32) | (qi < 32)
                    masked.append(jnp.where(mask, sc, -1e9))
                scores = jnp.concatenate(masked, axis=1)
                m = jnp.max(scores, axis=1, keepdims=True)
                w = jnp.exp(scores - m)
                w = w / jnp.sum(w, axis=1, keepdims=True)
                ctx = jnp.zeros((SQ, DH), jnp.float32)
                for s in range(N_DEV):
                    v_chunk = v_ref[b, h] if s == 0 else v_all[s - 1, b, h]
                    wb = w[:, s * SKV_SHARD:(s + 1) * SKV_SHARD].astype(jnp.bfloat16)
                    ctx = ctx + lax.dot_general(
                        wb, v_chunk, (((1,), (0,)), ((), ())),
                        preferred_element_type=jnp.float32,
                    )
                head_ctxs.append(ctx)
            ctx_b = jnp.concatenate(head_ctxs, axis=1).astype(jnp.bfloat16)
            out_ref[b] = lax.dot_general(
                ctx_b, wo_ref[...], (((1,), (0,)), ((), ())),
                preferred_element_type=jnp.float32,
            )

    return pl.pallas_call(
        body,
        out_shape=jax.ShapeDtypeStruct((B, SQ, DM), jnp.float32),
        in_specs=[pl.BlockSpec(memory_space=pltpu.VMEM)] * 5,
        out_specs=pl.BlockSpec(memory_space=pltpu.VMEM),
        scratch_shapes=[
            pltpu.VMEM((N_DEV - 1, B, HQ, SKV_SHARD, DH), jnp.bfloat16),
            pltpu.VMEM((N_DEV - 1, B, HQ, SKV_SHARD, DH), jnp.bfloat16),
            pltpu.SemaphoreType.DMA((N_DEV - 1,)),
            pltpu.SemaphoreType.DMA((N_DEV - 1,)),
            pltpu.SemaphoreType.DMA((N_DEV - 1,)),
            pltpu.SemaphoreType.DMA((N_DEV - 1,)),
        ],
        compiler_params=pltpu.CompilerParams(collective_id=0),
    )(xb, wq, kt, vt, wo)


# device time: 23771 ns/iter; 1.4137x vs baseline; 1.4137x over previous
import jax
import jax.numpy as jnp
from jax import lax
from jax.experimental import pallas as pl
from jax.experimental.pallas import tpu as pltpu

N_DEV = 4
B, SQ, SKV, HQ, DH = 2, 128, 512, 4, 64
SKV_SHARD = SKV // N_DEV
DM = 512
SCALE = 0.125


def kernel(x, Wq, K_ext, V_ext, Wo):
    xb = x.astype(jnp.bfloat16)
    wq = Wq.astype(jnp.bfloat16)
    wo = Wo.astype(jnp.bfloat16)
    kt = jnp.transpose(K_ext, (0, 2, 1, 3)).astype(jnp.bfloat16)
    vt = jnp.transpose(V_ext, (0, 2, 1, 3)).astype(jnp.bfloat16)

    def body(x_ref, wq_ref, k_ref, v_ref, wo_ref, out_ref,
             y_send, s_send, y_all, s_all,
             ysend_sems, yrecv_sems, ssend_sems, srecv_sems):
        my_pos = lax.axis_index("i")

        barrier = pltpu.get_barrier_semaphore()
        for r in (1, 2, 3):
            pl.semaphore_signal(
                barrier, inc=1,
                device_id=((my_pos + r) % N_DEV,),
                device_id_type=pl.DeviceIdType.MESH,
            )
        pl.semaphore_wait(barrier, N_DEV - 1)

        qi = lax.broadcasted_iota(jnp.int32, (SQ, SKV_SHARD), 0)
        kj = lax.broadcasted_iota(jnp.int32, (SQ, SKV_SHARD), 1) \
            + my_pos * SKV_SHARD
        mask = (jnp.abs(qi - kj) <= 128) | (kj < 32) | (qi < 32)
        bias = jnp.where(mask, 0.0, -1e9)

        own_y = [[None] * HQ for _ in range(B)]
        own_s = [[None] * HQ for _ in range(B)]
        for b in range(B):
            q_b = lax.dot_general(
                x_ref[b], wq_ref[...], (((1,), (0,)), ((), ())),
                preferred_element_type=jnp.float32,
            )
            for h in range(HQ):
                q_bh = (q_b[:, h * DH:(h + 1) * DH] * SCALE).astype(jnp.bfloat16)
                sc = lax.dot_general(
                    q_bh, k_ref[b, h], (((1,), (1,)), ((), ())),
                    preferred_element_type=jnp.float32,
                ) + bias
                w = jnp.exp(sc)
                den = jnp.sum(w, axis=1, keepdims=True)
                y = lax.dot_general(
                    w.astype(jnp.bfloat16), v_ref[b, h],
                    (((1,), (0,)), ((), ())),
                    preferred_element_type=jnp.float32,
                )
                own_y[b][h] = y
                own_s[b][h] = den
                y_send[b, h] = y.astype(jnp.bfloat16)
                s_send[b, h] = den

        rdmas = []
        for r in (1, 2, 3):
            p = (my_pos + r) % N_DEV
            slot = 3 - r
            ry = pltpu.make_async_remote_copy(
                src_ref=y_send, dst_ref=y_all.at[slot],
                send_sem=ysend_sems.at[slot], recv_sem=yrecv_sems.at[slot],
                device_id=(p,), device_id_type=pl.DeviceIdType.MESH,
            )
            rs = pltpu.make_async_remote_copy(
                src_ref=s_send, dst_ref=s_all.at[slot],
                send_sem=ssend_sems.at[slot], recv_sem=srecv_sems.at[slot],
                device_id=(p,), device_id_type=pl.DeviceIdType.MESH,
            )
            ry.start()
            rs.start()
            rdmas.append((ry, rs))

        for ry, rs in rdmas:
            ry.wait_recv()
            rs.wait_recv()
        for ry, rs in rdmas:
            ry.wait_send()
            rs.wait_send()

        for b in range(B):
            hctx = []
            for h in range(HQ):
                num = own_y[b][h]
                den = own_s[b][h]
                for s in range(N_DEV - 1):
                    num = num + y_all[s, b, h].astype(jnp.float32)
                    den = den + s_all[s, b, h]
                hctx.append(num / den)
            ctx_b = jnp.concatenate(hctx, axis=1).astype(jnp.bfloat16)
            out_ref[b] = lax.dot_general(
                ctx_b, wo_ref[...], (((1,), (0,)), ((), ())),
                preferred_element_type=jnp.float32,
            )

    return pl.pallas_call(
        body,
        out_shape=jax.ShapeDtypeStruct((B, SQ, DM), jnp.float32),
        in_specs=[pl.BlockSpec(memory_space=pltpu.VMEM)] * 5,
        out_specs=pl.BlockSpec(memory_space=pltpu.VMEM),
        scratch_shapes=[
            pltpu.VMEM((B, HQ, SQ, DH), jnp.bfloat16),
            pltpu.VMEM((B, HQ, SQ, 1), jnp.float32),
            pltpu.VMEM((N_DEV - 1, B, HQ, SQ, DH), jnp.bfloat16),
            pltpu.VMEM((N_DEV - 1, B, HQ, SQ, 1), jnp.float32),
            pltpu.SemaphoreType.DMA((N_DEV - 1,)),
            pltpu.SemaphoreType.DMA((N_DEV - 1,)),
            pltpu.SemaphoreType.DMA((N_DEV - 1,)),
            pltpu.SemaphoreType.DMA((N_DEV - 1,)),
        ],
        compiler_params=pltpu.CompilerParams(collective_id=0),
    )(xb, wq, kt, vt, wo)


# device time: 18815 ns/iter; 1.7861x vs baseline; 1.2634x over previous
import jax
import jax.numpy as jnp
from jax import lax
from jax.experimental import pallas as pl
from jax.experimental.pallas import tpu as pltpu

N_DEV = 4
B, SQ, SKV, HQ, DH = 2, 128, 512, 4, 64
SKV_SHARD = SKV // N_DEV
DM = 512
DP = 128
SCALE = 0.125


def kernel(x, Wq, K_ext, V_ext, Wo):
    xb = x.astype(jnp.bfloat16)
    wq = Wq.astype(jnp.bfloat16)
    wo = Wo.astype(jnp.bfloat16)
    kt = jnp.transpose(K_ext, (0, 2, 1, 3)).astype(jnp.bfloat16)
    vt = jnp.transpose(V_ext, (0, 2, 1, 3)).astype(jnp.bfloat16)
    ones = jnp.ones((B, HQ, SKV_SHARD, 1), jnp.bfloat16)
    zeros = jnp.zeros((B, HQ, SKV_SHARD, DP - DH - 1), jnp.bfloat16)
    v_pad = jnp.concatenate([vt, ones, zeros], axis=-1)

    def body(x_ref, wq_ref, k_ref, v_ref, wo_ref, out_ref,
             y_send, y_all, send_sems, recv_sems):
        my_pos = lax.axis_index("i")

        barrier = pltpu.get_barrier_semaphore()
        for r in (1, 2, 3):
            pl.semaphore_signal(
                barrier, inc=1,
                device_id=((my_pos + r) % N_DEV,),
                device_id_type=pl.DeviceIdType.MESH,
            )
        pl.semaphore_wait(barrier, N_DEV - 1)

        qi = lax.broadcasted_iota(jnp.int32, (SQ, SKV_SHARD), 0)
        kj = lax.broadcasted_iota(jnp.int32, (SQ, SKV_SHARD), 1) \
            + my_pos * SKV_SHARD
        mask = (jnp.abs(qi - kj) <= 128) | (kj < 32) | (qi < 32)
        bias = jnp.where(mask, 0.0, -1e9)

        own = [[None] * HQ for _ in range(B)]
        for b in range(B):
            q_b = lax.dot_general(
                x_ref[b], wq_ref[...], (((1,), (0,)), ((), ())),
                preferred_element_type=jnp.float32,
            )
            for h in range(HQ):
                q_bh = (q_b[:, h * DH:(h + 1) * DH] * SCALE).astype(jnp.bfloat16)
                sc = lax.dot_general(
                    q_bh, k_ref[b, h], (((1,), (1,)), ((), ())),
                    preferred_element_type=jnp.float32,
                ) + bias
                w = jnp.exp(sc).astype(jnp.bfloat16)
                y = lax.dot_general(
                    w, v_ref[b, h], (((1,), (0,)), ((), ())),
                    preferred_element_type=jnp.float32,
                )
                own[b][h] = y
                y_send[b, h] = y.astype(jnp.bfloat16)

        rdmas = []
        for r in (1, 2, 3):
            p = (my_pos + r) % N_DEV
            slot = 3 - r
            ry = pltpu.make_async_remote_copy(
                src_ref=y_send, dst_ref=y_all.at[slot],
                send_sem=send_sems.at[slot], recv_sem=recv_sems.at[slot],
                device_id=(p,), device_id_type=pl.DeviceIdType.MESH,
            )
            ry.start()
            rdmas.append(ry)

        for ry in rdmas:
            ry.wait_recv()
        for ry in rdmas:
            ry.wait_send()

        for b in range(B):
            hctx = []
            for h in range(HQ):
                tot = own[b][h]
                for s in range(N_DEV - 1):
                    tot = tot + y_all[s, b, h].astype(jnp.float32)
                hctx.append(tot[:, :DH] / tot[:, DH:DH + 1])
            ctx_b = jnp.concatenate(hctx, axis=1).astype(jnp.bfloat16)
            out_ref[b] = lax.dot_general(
                ctx_b, wo_ref[...], (((1,), (0,)), ((), ())),
                preferred_element_type=jnp.float32,
            )

    return pl.pallas_call(
        body,
        out_shape=jax.ShapeDtypeStruct((B, SQ, DM), jnp.float32),
        in_specs=[pl.BlockSpec(memory_space=pltpu.VMEM)] * 5,
        out_specs=pl.BlockSpec(memory_space=pltpu.VMEM),
        scratch_shapes=[
            pltpu.VMEM((B, HQ, SQ, DP), jnp.bfloat16),
            pltpu.VMEM((N_DEV - 1, B, HQ, SQ, DP), jnp.bfloat16),
            pltpu.SemaphoreType.DMA((N_DEV - 1,)),
            pltpu.SemaphoreType.DMA((N_DEV - 1,)),
        ],
        compiler_params=pltpu.CompilerParams(collective_id=0),
    )(xb, wq, kt, v_pad, wo)


# device time: 18057 ns/iter; 1.8611x vs baseline; 1.0420x over previous
import jax
import jax.numpy as jnp
from jax import lax
from jax.experimental import pallas as pl
from jax.experimental.pallas import tpu as pltpu

N_DEV = 4
B, SQ, SKV, HQ, DH = 2, 128, 512, 4, 64
SKV_SHARD = SKV // N_DEV
DM = 512
DP = 128
SQ_GLOBAL = 32
SCALE = 0.125


def kernel(x, Wq, K_ext, V_ext, Wo):
    xb = x.astype(jnp.bfloat16)
    wq = Wq.astype(jnp.bfloat16)
    wo = Wo.astype(jnp.bfloat16)
    kt = jnp.transpose(K_ext, (0, 2, 1, 3)).astype(jnp.bfloat16)
    vt = jnp.transpose(V_ext, (0, 2, 1, 3)).astype(jnp.bfloat16)
    ones = jnp.ones((B, HQ, SKV_SHARD, 1), jnp.bfloat16)
    zeros = jnp.zeros((B, HQ, SKV_SHARD, DP - DH - 1), jnp.bfloat16)
    v_pad = jnp.concatenate([vt, ones, zeros], axis=-1)

    def body(x_ref, wq_ref, k_ref, v_ref, wo_ref, out_ref,
             y_send, y_all, send_sems, recv_sems):
        my_pos = lax.axis_index("i")

        for s in range(N_DEV - 1):
            y_all[s, :, :, SQ_GLOBAL:, :] = jnp.zeros(
                (B, HQ, SQ - SQ_GLOBAL, DP), jnp.bfloat16)

        qi = lax.broadcasted_iota(jnp.int32, (SQ, SKV_SHARD), 0)
        kj = lax.broadcasted_iota(jnp.int32, (SQ, SKV_SHARD), 1) \
            + my_pos * SKV_SHARD
        mask = (jnp.abs(qi - kj) <= 128) | (kj < 32) | (qi < 32)
        bias = jnp.where(mask, 0.0, -1e9)

        own = [[None] * HQ for _ in range(B)]
        for b in range(B):
            q_b = lax.dot_general(
                x_ref[b], wq_ref[...], (((1,), (0,)), ((), ())),
                preferred_element_type=jnp.float32,
            )
            for h in range(HQ):
                q_bh = (q_b[:, h * DH:(h + 1) * DH] * SCALE).astype(jnp.bfloat16)
                sc = lax.dot_general(
                    q_bh, k_ref[b, h], (((1,), (1,)), ((), ())),
                    preferred_element_type=jnp.float32,
                ) + bias
                w = jnp.exp(sc).astype(jnp.bfloat16)
                y = lax.dot_general(
                    w, v_ref[b, h], (((1,), (0,)), ((), ())),
                    preferred_element_type=jnp.float32,
                )
                own[b][h] = y
                y_send[b, h] = y.astype(jnp.bfloat16)

        barrier = pltpu.get_barrier_semaphore()
        for r in (1, 2, 3):
            pl.semaphore_signal(
                barrier, inc=1,
                device_id=((my_pos + r) % N_DEV,),
                device_id_type=pl.DeviceIdType.MESH,
            )
        pl.semaphore_wait(barrier, N_DEV - 1)

        send_full, send_part = [], []
        for r in (1, 2, 3):
            p = (my_pos + r) % N_DEV
            slot = 3 - r
            full = pltpu.make_async_remote_copy(
                src_ref=y_send, dst_ref=y_all.at[slot],
                send_sem=send_sems.at[slot], recv_sem=recv_sems.at[slot],
                device_id=(p,), device_id_type=pl.DeviceIdType.MESH,
            )
            part = pltpu.make_async_remote_copy(
                src_ref=y_send.at[:, :, pl.ds(0, SQ_GLOBAL), :],
                dst_ref=y_all.at[slot, :, :, pl.ds(0, SQ_GLOBAL), :],
                send_sem=send_sems.at[slot], recv_sem=recv_sems.at[slot],
                device_id=(p,), device_id_type=pl.DeviceIdType.MESH,
            )
            send_full.append(full)
            send_part.append(part)

            @pl.when(my_pos < 2)
            def _(full=full):
                full.start()

            @pl.when(my_pos >= 2)
            def _(part=part):
                part.start()

        for s in range(N_DEV - 1):
            origin = (my_pos + s + 1) % N_DEV
            fullw = pltpu.make_async_remote_copy(
                src_ref=y_send, dst_ref=y_all.at[s],
                send_sem=send_sems.at[s], recv_sem=recv_sems.at[s],
                device_id=(my_pos,), device_id_type=pl.DeviceIdType.MESH,
            )
            partw = pltpu.make_async_remote_copy(
                src_ref=y_send.at[:, :, pl.ds(0, SQ_GLOBAL), :],
                dst_ref=y_all.at[s, :, :, pl.ds(0, SQ_GLOBAL), :],
                send_sem=send_sems.at[s], recv_sem=recv_sems.at[s],
                device_id=(my_pos,), device_id_type=pl.DeviceIdType.MESH,
            )

            @pl.when(origin < 2)
            def _(fullw=fullw):
                fullw.wait_recv()

            @pl.when(origin >= 2)
            def _(partw=partw):
                partw.wait_recv()

        for i in range(N_DEV - 1):
            @pl.when(my_pos < 2)
            def _(d=send_full[i]):
                d.wait_send()

            @pl.when(my_pos >= 2)
            def _(d=send_part[i]):
                d.wait_send()

        for b in range(B):
            hctx = []
            for h in range(HQ):
                tot = own[b][h]
                for s in range(N_DEV - 1):
                    tot = tot + y_all[s, b, h].astype(jnp.float32)
                hctx.append(tot[:, :DH] / tot[:, DH:DH + 1])
            ctx_b = jnp.concatenate(hctx, axis=1).astype(jnp.bfloat16)
            out_ref[b] = lax.dot_general(
                ctx_b, wo_ref[...], (((1,), (0,)), ((), ())),
                preferred_element_type=jnp.float32,
            )

    return pl.pallas_call(
        body,
        out_shape=jax.ShapeDtypeStruct((B, SQ, DM), jnp.float32),
        in_specs=[pl.BlockSpec(memory_space=pltpu.VMEM)] * 5,
        out_specs=pl.BlockSpec(memory_space=pltpu.VMEM),
        scratch_shapes=[
            pltpu.VMEM((B, HQ, SQ, DP), jnp.bfloat16),
            pltpu.VMEM((N_DEV - 1, B, HQ, SQ, DP), jnp.bfloat16),
            pltpu.SemaphoreType.DMA((N_DEV - 1,)),
            pltpu.SemaphoreType.DMA((N_DEV - 1,)),
        ],
        compiler_params=pltpu.CompilerParams(collective_id=0),
    )(xb, wq, kt, v_pad, wo)


# device time: 13738 ns/iter; 2.4461x vs baseline; 1.3144x over previous
import jax
import jax.numpy as jnp
from jax import lax
from jax.experimental import pallas as pl
from jax.experimental.pallas import tpu as pltpu

N_DEV = 4
B, SQ, SKV, HQ, DH = 2, 128, 512, 4, 64
SKV_SHARD = SKV // N_DEV
DM = 512
NPAIR = HQ // 2
NBLK = B * NPAIR + 1
SQ_GLOBAL = 32
SCALE = 0.125


def kernel(x, Wq, K_ext, V_ext, Wo):
    xb = x.astype(jnp.bfloat16)
    wq = Wq.astype(jnp.bfloat16)
    wo = Wo.astype(jnp.bfloat16)
    kt = jnp.transpose(K_ext, (0, 2, 1, 3)).astype(jnp.bfloat16)
    vt = jnp.transpose(V_ext, (0, 2, 1, 3)).astype(jnp.bfloat16)
    z = jnp.zeros((B, HQ, SKV_SHARD, DH), jnp.bfloat16)
    v_even = jnp.concatenate([vt, z], axis=-1)
    v_odd = jnp.concatenate([z, vt], axis=-1)

    def body(x_ref, wq_ref, k_ref, ve_ref, vo_ref, wo_ref, out_ref,
             y_send, y_all, send_sems, recv_sems):
        my_pos = lax.axis_index("i")

        for s in range(N_DEV - 1):
            y_all[s, :, SQ_GLOBAL:, :] = jnp.zeros(
                (NBLK, SQ - SQ_GLOBAL, 128), jnp.bfloat16)

        qi = lax.broadcasted_iota(jnp.int32, (SQ, SKV_SHARD), 0)
        kj = lax.broadcasted_iota(jnp.int32, (SQ, SKV_SHARD), 1) \
            + my_pos * SKV_SHARD
        mask = (jnp.abs(qi - kj) <= 128) | (kj < 32) | (qi < 32)
        bias = jnp.where(mask, 0.0, -1e9)
        lane = lax.broadcasted_iota(jnp.int32, (SKV_SHARD, 128), 1)

        own = [None] * NBLK
        den_tile = jnp.zeros((SQ, 128), jnp.float32)
        for b in range(B):
            q_b = lax.dot_general(
                x_ref[b], wq_ref[...], (((1,), (0,)), ((), ())),
                preferred_element_type=jnp.float32,
            )
            ws = []
            for h in range(HQ):
                q_bh = (q_b[:, h * DH:(h + 1) * DH] * SCALE).astype(jnp.bfloat16)
                sc = lax.dot_general(
                    q_bh, k_ref[b, h], (((1,), (1,)), ((), ())),
                    preferred_element_type=jnp.float32,
                ) + bias
                w = jnp.exp(sc).astype(jnp.bfloat16)
                ws.append(w)
                e_col = jnp.where(lane == b * HQ + h, 1.0, 0.0).astype(jnp.bfloat16)
                den_tile = den_tile + lax.dot_general(
                    w, e_col, (((1,), (0,)), ((), ())),
                    preferred_element_type=jnp.float32,
                )
            for p in range(NPAIR):
                y_pair = lax.dot_general(
                    ws[2 * p], ve_ref[b, 2 * p], (((1,), (0,)), ((), ())),
                    preferred_element_type=jnp.float32,
                ) + lax.dot_general(
                    ws[2 * p + 1], vo_ref[b, 2 * p + 1], (((1,), (0,)), ((), ())),
                    preferred_element_type=jnp.float32,
                )
                blk = b * NPAIR + p
                own[blk] = y_pair
                y_send[blk] = y_pair.astype(jnp.bfloat16)
        own[NBLK - 1] = den_tile
        y_send[NBLK - 1] = den_tile.astype(jnp.bfloat16)

        barrier = pltpu.get_barrier_semaphore()
        for r in (1, 2, 3):
            pl.semaphore_signal(
                barrier, inc=1,
                device_id=((my_pos + r) % N_DEV,),
                device_id_type=pl.DeviceIdType.MESH,
            )
        pl.semaphore_wait(barrier, N_DEV - 1)

        send_full, send_part = [], []
        for r in (1, 2, 3):
            p = (my_pos + r) % N_DEV
            slot = 3 - r
            full = pltpu.make_async_remote_copy(
                src_ref=y_send, dst_ref=y_all.at[slot],
                send_sem=send_sems.at[slot], recv_sem=recv_sems.at[slot],
                device_id=(p,), device_id_type=pl.DeviceIdType.MESH,
            )
            part = pltpu.make_async_remote_copy(
                src_ref=y_send.at[:, pl.ds(0, SQ_GLOBAL), :],
                dst_ref=y_all.at[slot, :, pl.ds(0, SQ_GLOBAL), :],
                send_sem=send_sems.at[slot], recv_sem=recv_sems.at[slot],
                device_id=(p,), device_id_type=pl.DeviceIdType.MESH,
            )
            send_full.append(full)
            send_part.append(part)

            @pl.when(my_pos < 2)
            def _(full=full):
                full.start()

            @pl.when(my_pos >= 2)
            def _(part=part):
                part.start()

        for s in range(N_DEV - 1):
            origin = (my_pos + s + 1) % N_DEV
            fullw = pltpu.make_async_remote_copy(
                src_ref=y_send, dst_ref=y_all.at[s],
                send_sem=send_sems.at[s], recv_sem=recv_sems.at[s],
                device_id=(my_pos,), device_id_type=pl.DeviceIdType.MESH,
            )
            partw = pltpu.make_async_remote_copy(
                src_ref=y_send.at[:, pl.ds(0, SQ_GLOBAL), :],
                dst_ref=y_all.at[s, :, pl.ds(0, SQ_GLOBAL), :],
                send_sem=send_sems.at[s], recv_sem=recv_sems.at[s],
                device_id=(my_pos,), device_id_type=pl.DeviceIdType.MESH,
            )

            @pl.when(origin < 2)
            def _(fullw=fullw):
                fullw.wait_recv()

            @pl.when(origin >= 2)
            def _(partw=partw):
                partw.wait_recv()

        for i in range(N_DEV - 1):
            @pl.when(my_pos < 2)
            def _(d=send_full[i]):
                d.wait_send()

            @pl.when(my_pos >= 2)
            def _(d=send_part[i]):
                d.wait_send()

        tot_den = own[NBLK - 1]
        for s in range(N_DEV - 1):
            tot_den = tot_den + y_all[s, NBLK - 1].astype(jnp.float32)
        lane_sq = lax.broadcasted_iota(jnp.int32, (SQ, 128), 1)
        for b in range(B):
            pctx = []
            for p in range(NPAIR):
                blk = b * NPAIR + p
                tot = own[blk]
                for s in range(N_DEV - 1):
                    tot = tot + y_all[s, blk].astype(jnp.float32)
                d_even = tot_den[:, b * HQ + 2 * p:b * HQ + 2 * p + 1]
                d_odd = tot_den[:, b * HQ + 2 * p + 1:b * HQ + 2 * p + 2]
                divisor = jnp.where(lane_sq < DH, d_even, d_odd)
                pctx.append(tot / divisor)
            ctx_b = jnp.concatenate(pctx, axis=1).astype(jnp.bfloat16)
            out_ref[b] = lax.dot_general(
                ctx_b, wo_ref[...], (((1,), (0,)), ((), ())),
                preferred_element_type=jnp.float32,
            )

    return pl.pallas_call(
        body,
        out_shape=jax.ShapeDtypeStruct((B, SQ, DM), jnp.float32),
        in_specs=[pl.BlockSpec(memory_space=pltpu.VMEM)] * 6,
        out_specs=pl.BlockSpec(memory_space=pltpu.VMEM),
        scratch_shapes=[
            pltpu.VMEM((NBLK, SQ, 128), jnp.bfloat16),
            pltpu.VMEM((N_DEV - 1, NBLK, SQ, 128), jnp.bfloat16),
            pltpu.SemaphoreType.DMA((N_DEV - 1,)),
            pltpu.SemaphoreType.DMA((N_DEV - 1,)),
        ],
        compiler_params=pltpu.CompilerParams(collective_id=0),
    )(xb, wq, kt, v_even, v_odd, wo)


# device time: 1987 ns/iter; 16.9124x vs baseline; 6.9139x over previous
import jax
import jax.numpy as jnp
from jax import lax
from jax.experimental import pallas as pl
from jax.experimental.pallas import tpu as pltpu

N_DEV = 4
B, SQ, SKV, HQ, DH = 2, 128, 512, 4, 64
SKV_SHARD = SKV // N_DEV
DM = 512
NPAIR = HQ // 2
NBLK = B * NPAIR + 1
SQ_GLOBAL = 32
SCALE = 0.125


def kernel(x, Wq, K_ext, V_ext, Wo):
    kr = K_ext.reshape(B, SKV_SHARD, HQ * DH)
    vr = V_ext.reshape(B, SKV_SHARD, HQ * DH)

    def body(x_ref, wq_ref, k_ref, v_ref, wo_ref, out_ref,
             y_send, y_all, send_sems, recv_sems):
        my_pos = lax.axis_index("i")

        for s in range(N_DEV - 1):
            y_all[s, :, SQ_GLOBAL:, :] = jnp.zeros(
                (NBLK, SQ - SQ_GLOBAL, 128), jnp.bfloat16)

        qi = lax.broadcasted_iota(jnp.int32, (SQ, SKV_SHARD), 0)
        kj = lax.broadcasted_iota(jnp.int32, (SQ, SKV_SHARD), 1) \
            + my_pos * SKV_SHARD
        mask = (jnp.abs(qi - kj) <= 128) | (kj < 32) | (qi < 32)
        bias = jnp.where(mask, 0.0, -1e9)
        lane = lax.broadcasted_iota(jnp.int32, (SKV_SHARD, 128), 1)

        wq_b = wq_ref[...].astype(jnp.bfloat16)

        own = [None] * NBLK
        den_tile = jnp.zeros((SQ, 128), jnp.float32)
        for b in range(B):
            k_slab = k_ref[b].astype(jnp.bfloat16)
            v_slab = v_ref[b].astype(jnp.bfloat16)
            q_b = lax.dot_general(
                x_ref[b].astype(jnp.bfloat16), wq_b,
                (((1,), (0,)), ((), ())),
                preferred_element_type=jnp.float32,
            )
            ws = []
            for h in range(HQ):
                q_bh = (q_b[:, h * DH:(h + 1) * DH] * SCALE).astype(jnp.bfloat16)
                sc = lax.dot_general(
                    q_bh, k_slab[:, h * DH:(h + 1) * DH],
                    (((1,), (1,)), ((), ())),
                    preferred_element_type=jnp.float32,
                ) + bias
                w = jnp.exp(sc).astype(jnp.bfloat16)
                ws.append(w)
                e_col = jnp.where(lane == b * HQ + h, 1.0, 0.0).astype(jnp.bfloat16)
                den_tile = den_tile + lax.dot_general(
                    w, e_col, (((1,), (0,)), ((), ())),
                    preferred_element_type=jnp.float32,
                )
            for p in range(NPAIR):
                pair_slab = v_slab[:, p * 128:(p + 1) * 128]
                ve = jnp.where(lane < DH, pair_slab, 0).astype(jnp.bfloat16)
                vo = jnp.where(lane >= DH, pair_slab, 0).astype(jnp.bfloat16)
                y_pair = lax.dot_general(
                    ws[2 * p], ve, (((1,), (0,)), ((), ())),
                    preferred_element_type=jnp.float32,
                ) + lax.dot_general(
                    ws[2 * p + 1], vo, (((1,), (0,)), ((), ())),
                    preferred_element_type=jnp.float32,
                )
                blk = b * NPAIR + p
                own[blk] = y_pair
                y_send[blk] = y_pair.astype(jnp.bfloat16)
        own[NBLK - 1] = den_tile
        y_send[NBLK - 1] = den_tile.astype(jnp.bfloat16)

        barrier = pltpu.get_barrier_semaphore()
        for r in (1, 2, 3):
            pl.semaphore_signal(
                barrier, inc=1,
                device_id=((my_pos + r) % N_DEV,),
                device_id_type=pl.DeviceIdType.MESH,
            )
        pl.semaphore_wait(barrier, N_DEV - 1)

        send_full, send_part = [], []
        for r in (1, 2, 3):
            p = (my_pos + r) % N_DEV
            slot = 3 - r
            full = pltpu.make_async_remote_copy(
                src_ref=y_send, dst_ref=y_all.at[slot],
                send_sem=send_sems.at[slot], recv_sem=recv_sems.at[slot],
                device_id=(p,), device_id_type=pl.DeviceIdType.MESH,
            )
            part = pltpu.make_async_remote_copy(
                src_ref=y_send.at[:, pl.ds(0, SQ_GLOBAL), :],
                dst_ref=y_all.at[slot, :, pl.ds(0, SQ_GLOBAL), :],
                send_sem=send_sems.at[slot], recv_sem=recv_sems.at[slot],
                device_id=(p,), device_id_type=pl.DeviceIdType.MESH,
            )
            send_full.append(full)
            send_part.append(part)

            @pl.when(my_pos < 2)
            def _(full=full):
                full.start()

            @pl.when(my_pos >= 2)
            def _(part=part):
                part.start()

        tot = list(own)
        for s in (0, 2, 1):
            origin = (my_pos + s + 1) % N_DEV
            fullw = pltpu.make_async_remote_copy(
                src_ref=y_send, dst_ref=y_all.at[s],
                send_sem=send_sems.at[s], recv_sem=recv_sems.at[s],
                device_id=(my_pos,), device_id_type=pl.DeviceIdType.MESH,
            )
            partw = pltpu.make_async_remote_copy(
                src_ref=y_send.at[:, pl.ds(0, SQ_GLOBAL), :],
                dst_ref=y_all.at[s, :, pl.ds(0, SQ_GLOBAL), :],
                send_sem=send_sems.at[s], recv_sem=recv_sems.at[s],
                device_id=(my_pos,), device_id_type=pl.DeviceIdType.MESH,
            )

            @pl.when(origin < 2)
            def _(fullw=fullw):
                fullw.wait_recv()

            @pl.when(origin >= 2)
            def _(partw=partw):
                partw.wait_recv()

            for blk in range(NBLK):
                tot[blk] = tot[blk] + y_all[s, blk].astype(jnp.float32)

        tot_den = tot[NBLK - 1]
        lane_sq = lax.broadcasted_iota(jnp.int32, (SQ, 128), 1)
        wo_b = wo_ref[...].astype(jnp.bfloat16)
        for b in range(B):
            pctx = []
            for p in range(NPAIR):
                blk = b * NPAIR + p
                d_even = tot_den[:, b * HQ + 2 * p:b * HQ + 2 * p + 1]
                d_odd = tot_den[:, b * HQ + 2 * p + 1:b * HQ + 2 * p + 2]
                divisor = jnp.where(lane_sq < DH, d_even, d_odd)
                pctx.append(tot[blk] / divisor)
            ctx_b = jnp.concatenate(pctx, axis=1).astype(jnp.bfloat16)
            out_ref[b] = lax.dot_general(
                ctx_b, wo_b, (((1,), (0,)), ((), ())),
                preferred_element_type=jnp.float32,
            )

        for i in range(N_DEV - 1):
            @pl.when(my_pos < 2)
            def _(d=send_full[i]):
                d.wait_send()

            @pl.when(my_pos >= 2)
            def _(d=send_part[i]):
                d.wait_send()

    return pl.pallas_call(
        body,
        out_shape=jax.ShapeDtypeStruct((B, SQ, DM), jnp.float32),
        in_specs=[pl.BlockSpec(memory_space=pltpu.VMEM)] * 5,
        out_specs=pl.BlockSpec(memory_space=pltpu.VMEM),
        scratch_shapes=[
            pltpu.VMEM((NBLK, SQ, 128), jnp.bfloat16),
            pltpu.VMEM((N_DEV - 1, NBLK, SQ, 128), jnp.bfloat16),
            pltpu.SemaphoreType.DMA((N_DEV - 1,)),
            pltpu.SemaphoreType.DMA((N_DEV - 1,)),
        ],
        compiler_params=pltpu.CompilerParams(collective_id=0),
    )(x, Wq, kr, vr, Wo)
